# baseline (speedup 1.0000x reference)
import jax
import jax.numpy as jnp
from jax import lax
from jax.experimental import pallas as pl
from jax.experimental.pallas import tpu as pltpu
from jax.experimental.pallas import tpu_sc as plsc

N = 10000
NPAD = 10240
F = 128
H = 64
E = 320000

NC = 2
NS = 16
NW = NC * NS
B = 96
EPW = E // NW
EPAD = 10080
NB = EPAD // B
RPT = NPAD // NS
CO = 80
CH = RPT // CO
GB = 5
HR = NPAD // 128
HT = HR // NS


def _pre_body(x_ref, wl_ref, wr_ref, b_ref, y_ref, z_ref):
    x = x_ref[...]
    y_ref[0:N] = jnp.dot(x, wl_ref[...], preferred_element_type=jnp.float32)
    y_ref[N:NPAD] = jnp.zeros((NPAD - N, H), jnp.float32)
    z_ref[...] = jnp.dot(x, wr_ref[...], preferred_element_type=jnp.float32) + b_ref[...]


def _inv_cnt(cntp_ref):
    cnt = jnp.maximum(cntp_ref[0:1, 0:N] + cntp_ref[1:2, 0:N], 1.0)
    return jnp.transpose(cnt)


def _mid_body(aggp_ref, cntp_ref, z1_ref, wl_ref, wr_ref, b_ref, y_ref, z_ref):
    agg = aggp_ref[0:N, 0:H] + aggp_ref[0:N, H:2 * H]
    h = jax.nn.sigmoid(agg / _inv_cnt(cntp_ref) + z1_ref[...])
    y_ref[0:N] = jnp.dot(h, wl_ref[...], preferred_element_type=jnp.float32)
    y_ref[N:NPAD] = jnp.zeros((NPAD - N, H), jnp.float32)
    z_ref[...] = jnp.dot(h, wr_ref[...], preferred_element_type=jnp.float32) + b_ref[...]


def _post_body(aggp_ref, cntp_ref, z2_ref, out_ref):
    agg = aggp_ref[0:N, 0:H] + aggp_ref[0:N, H:2 * H]
    out_ref[...] = agg / _inv_cnt(cntp_ref) + z2_ref[...]


_tc_pre = pl.pallas_call(
    _pre_body,
    out_shape=[jax.ShapeDtypeStruct((NPAD, H), jnp.float32),
               jax.ShapeDtypeStruct((N, H), jnp.float32)],
)

_tc_mid = pl.pallas_call(
    _mid_body,
    out_shape=[jax.ShapeDtypeStruct((NPAD, H), jnp.float32),
               jax.ShapeDtypeStruct((N, H), jnp.float32)],
)

_tc_post = pl.pallas_call(
    _post_body,
    out_shape=jax.ShapeDtypeStruct((N, H), jnp.float32),
)


def _fill_rows(buf, ncols, val):
    v = jnp.full((16,), val, jnp.float32)

    def body(i, _):
        for j in range(ncols // 16):
            buf[i, pl.ds(j * 16, 16)] = v
        return 0

    lax.fori_loop(0, buf.shape[0], body, 0)


def _make_sc(with_cnt):
    mesh = plsc.VectorSubcoreMesh(
        core_axis_name="c", subcore_axis_name="s", num_cores=NC, num_subcores=NS)
    out_type = [jax.ShapeDtypeStruct((NPAD, NC * H), jnp.float32)]
    scratch = (
        [pltpu.VMEM((NB, B), jnp.int32),
         pltpu.VMEM((NB, B), jnp.int32)]
        + [pltpu.VMEM((B, H), jnp.float32)] * GB
        + [pltpu.VMEM((CO, H), jnp.float32),
           pltpu.VMEM_SHARED((NPAD, H), jnp.float32)]
        + [pltpu.SemaphoreType.DMA] * GB
        + [pltpu.SemaphoreType.DMA] * GB
    )
    if with_cnt:
        out_type.append(jax.ShapeDtypeStruct((NC, HR, 128), jnp.float32))
        scratch = scratch + [
            pltpu.VMEM((HR, 128), jnp.float32),
            pltpu.VMEM((1, HR), jnp.int32),
            pltpu.VMEM((HT, 128), jnp.float32),
            pltpu.VMEM_SHARED((HR, 128), jnp.float32),
        ]

    def body(table, idx3, *refs):
        if with_cnt:
            (agg_out, cnt_out, srcv, dstv, *rest) = refs
        else:
            (agg_out, srcv, dstv, *rest) = refs
            cnt_out = None
        rows = rest[:GB]
        zrows, acc = rest[GB], rest[GB + 1]
        gsem = rest[GB + 2:2 * GB + 2]
        ssem = rest[2 * GB + 2:3 * GB + 2]
        if with_cnt:
            hist, rowid, cstage, cacc = rest[3 * GB + 2:]
        else:
            hist = rowid = cstage = cacc = None
        c = lax.axis_index("c")
        s = lax.axis_index("s")
        wid = c * NS + s
        base = s * RPT

        _fill_rows(zrows, H, 0.0)
        if with_cnt:
            _fill_rows(hist, 128, 0.0)
            for k in range(HR // 16):
                rowid[0, pl.ds(16 * k, 16)] = lax.iota(jnp.int32, 16) + 16 * k

            @pl.when(s == 0)
            def _():
                pltpu.sync_copy(hist, cacc)

        for k in range(CH):
            sl = pl.ds(base + k * CO, CO)
            pltpu.sync_copy(zrows, acc.at[sl])
        plsc.subcore_barrier()

        pltpu.sync_copy(idx3.at[0, wid], srcv)
        pltpu.sync_copy(idx3.at[1, wid], dstv)

        LEAD = GB - 2

        def wait_g(b, j):
            pltpu.make_async_copy(table.at[srcv.at[j]], rows[b], gsem[b]).wait()

        def wait_s(b):
            pltpu.make_async_copy(rows[b], acc.at[dstv.at[0]], ssem[b]).wait()

        def hist_batch(j):
            def hbody(k, _):
                d = dstv[j, pl.ds(16 * k, 16)]
                occ, last = plsc.scan_count(d)
                plsc.addupdate_scatter(
                    hist, [lax.shift_right_logical(d, 7),
                           lax.bitwise_and(d, 127)],
                    occ.astype(jnp.float32), mask=last)
                return 0

            lax.fori_loop(0, B // 16, hbody, 0)

        def issue(j, b, do_swait, do_gather):
            bw = (b + LEAD) % GB
            if do_swait:
                wait_s(bw)
            if do_gather:
                pltpu.async_copy(table.at[srcv.at[j + LEAD]], rows[bw], gsem[bw])
            wait_g(b, j)
            pltpu.async_copy(rows[b], acc.at[dstv.at[j]], ssem[b], add=True)
            if with_cnt:
                hist_batch(j)

        for b in range(LEAD):
            pltpu.async_copy(table.at[srcv.at[b]], rows[b], gsem[b])
        for b in range(GB):
            issue(b, b, do_swait=(b >= GB - LEAD), do_gather=True)

        def outer(t, _):
            for b in range(GB):
                issue(GB * t + b, b, do_swait=True, do_gather=True)
            return 0

        lax.fori_loop(1, NB // GB - 1, outer, 0)
        for b in range(GB):
            j = NB - GB + b
            issue(j, b, do_swait=True, do_gather=(j + LEAD < NB))
        for b in range(LEAD, GB):
            wait_s(b)
        if with_cnt:
            pltpu.sync_copy(hist, cacc.at[rowid.at[0]], add=True)
        plsc.subcore_barrier()

        for k in range(CH):
            sl = pl.ds(base + k * CO, CO)
            pltpu.sync_copy(acc.at[sl], zrows)
            pltpu.sync_copy(zrows, agg_out.at[sl, pl.ds(c * H, H)])
        if with_cnt:
            pltpu.sync_copy(cacc.at[pl.ds(s * HT, HT)], cstage)
            pltpu.sync_copy(cstage, cnt_out.at[c, pl.ds(s * HT, HT)])

    return pl.kernel(
        body, out_type=out_type, mesh=mesh, scratch_types=scratch,
        compiler_params=pltpu.CompilerParams(
            use_tc_tiling_on_sc=False,
            needs_layout_passes=False if with_cnt else None))


_sc_l1 = _make_sc(with_cnt=True)
_sc_l2 = _make_sc(with_cnt=False)


@jax.jit
def kernel(x, edge_index, W1l, b1l, W1r, W2l, b2l, W2r):
    e = edge_index.astype(jnp.int32).reshape(2, NW, EPW)
    pad = jnp.full((2, NW, EPAD - EPW), NPAD - 1, jnp.int32)
    idx3 = jnp.concatenate([e, pad], axis=2).reshape(2, NW, NB, B)

    y1, z1 = _tc_pre(x, W1l, W1r, b1l.reshape(1, H))
    agg1p, cntp = _sc_l1(y1, idx3)
    cnt2 = cntp.reshape(NC, NPAD)
    y2, z2 = _tc_mid(agg1p, cnt2, z1, W2l, W2r, b2l.reshape(1, H))
    [agg2p] = _sc_l2(y2, idx3)
    return _tc_post(agg2p, cnt2, z2)


# speedup vs baseline: 1.5505x; 1.5505x over previous
import jax
import jax.numpy as jnp
from jax import lax
from jax.experimental import pallas as pl
from jax.experimental.pallas import tpu as pltpu
from jax.experimental.pallas import tpu_sc as plsc

N = 10000
NPAD = 10240
F = 128
H = 64
E = 320000

NC = 2
NS = 16
NW = NC * NS
B = 80
EPW = E // NW
EPAD = EPW
NB = EPAD // B
RPT = NPAD // NS
CO = 80
CH = RPT // CO
GB = 5
HR = NPAD // 128
HT = HR // NS


def _pre_body(x_ref, wl_ref, wr_ref, b_ref, y_ref, z_ref):
    x = x_ref[...]
    y_ref[0:N] = jnp.dot(x, wl_ref[...], preferred_element_type=jnp.float32)
    y_ref[N:NPAD] = jnp.zeros((NPAD - N, H), jnp.float32)
    z_ref[...] = jnp.dot(x, wr_ref[...], preferred_element_type=jnp.float32) + b_ref[...]


def _inv_cnt(cntp_ref):
    cnt = jnp.maximum(cntp_ref[0:1, 0:N] + cntp_ref[1:2, 0:N], 1.0)
    return jnp.transpose(cnt)


def _mid_body(aggp_ref, cntp_ref, z1_ref, wl_ref, wr_ref, b_ref, y_ref, z_ref):
    agg = aggp_ref[0:N, 0:H] + aggp_ref[0:N, H:2 * H]
    h = jax.nn.sigmoid(agg / _inv_cnt(cntp_ref) + z1_ref[...])
    y_ref[0:N] = jnp.dot(h, wl_ref[...], preferred_element_type=jnp.float32)
    y_ref[N:NPAD] = jnp.zeros((NPAD - N, H), jnp.float32)
    z_ref[...] = jnp.dot(h, wr_ref[...], preferred_element_type=jnp.float32) + b_ref[...]


def _post_body(aggp_ref, cntp_ref, z2_ref, out_ref):
    agg = aggp_ref[0:N, 0:H] + aggp_ref[0:N, H:2 * H]
    out_ref[...] = agg / _inv_cnt(cntp_ref) + z2_ref[...]


_tc_pre = pl.pallas_call(
    _pre_body,
    out_shape=[jax.ShapeDtypeStruct((NPAD, H), jnp.float32),
               jax.ShapeDtypeStruct((N, H), jnp.float32)],
)

_tc_mid = pl.pallas_call(
    _mid_body,
    out_shape=[jax.ShapeDtypeStruct((NPAD, H), jnp.float32),
               jax.ShapeDtypeStruct((N, H), jnp.float32)],
)

_tc_post = pl.pallas_call(
    _post_body,
    out_shape=jax.ShapeDtypeStruct((N, H), jnp.float32),
)


def _fill_rows(buf, ncols, val):
    v = jnp.full((16,), val, jnp.float32)

    def body(i, _):
        for j in range(ncols // 16):
            buf[i, pl.ds(j * 16, 16)] = v
        return 0

    lax.fori_loop(0, buf.shape[0], body, 0)


def _make_sc(with_cnt):
    mesh = plsc.VectorSubcoreMesh(
        core_axis_name="c", subcore_axis_name="s", num_cores=NC, num_subcores=NS)
    out_type = [jax.ShapeDtypeStruct((NPAD, NC * H), jnp.float32)]
    scratch = (
        [pltpu.VMEM((NB, B), jnp.int32),
         pltpu.VMEM((NB, B), jnp.int32)]
        + [pltpu.VMEM((B, H), jnp.float32)] * GB
        + [pltpu.VMEM((CO, H), jnp.float32),
           pltpu.VMEM_SHARED((NPAD, H), jnp.float32)]
        + [pltpu.SemaphoreType.DMA] * GB
        + [pltpu.SemaphoreType.DMA] * GB
    )
    if with_cnt:
        out_type.append(jax.ShapeDtypeStruct((NC, HR, 128), jnp.float32))
        scratch = scratch + [
            pltpu.VMEM((HR, 128), jnp.float32),
            pltpu.VMEM((1, HR), jnp.int32),
            pltpu.VMEM((HT, 128), jnp.float32),
            pltpu.VMEM_SHARED((HR, 128), jnp.float32),
        ]

    def body(table, idx3, *refs):
        if with_cnt:
            (agg_out, cnt_out, srcv, dstv, *rest) = refs
        else:
            (agg_out, srcv, dstv, *rest) = refs
            cnt_out = None
        rows = rest[:GB]
        zrows, acc = rest[GB], rest[GB + 1]
        gsem = rest[GB + 2:2 * GB + 2]
        ssem = rest[2 * GB + 2:3 * GB + 2]
        if with_cnt:
            hist, rowid, cstage, cacc = rest[3 * GB + 2:]
        else:
            hist = rowid = cstage = cacc = None
        c = lax.axis_index("c")
        s = lax.axis_index("s")
        wid = c * NS + s
        base = s * RPT

        _fill_rows(zrows, H, 0.0)
        if with_cnt:
            _fill_rows(hist, 128, 0.0)
            for k in range(HR // 16):
                rowid[0, pl.ds(16 * k, 16)] = lax.iota(jnp.int32, 16) + 16 * k

            @pl.when(s == 0)
            def _():
                pltpu.sync_copy(hist, cacc)

        for k in range(CH):
            sl = pl.ds(base + k * CO, CO)
            pltpu.sync_copy(zrows, acc.at[sl])
        plsc.subcore_barrier()

        pltpu.sync_copy(idx3.at[0, wid], srcv)
        pltpu.sync_copy(idx3.at[1, wid], dstv)

        LEAD = GB - 2

        def wait_g(b, j):
            pltpu.make_async_copy(table.at[srcv.at[j]], rows[b], gsem[b]).wait()

        def wait_s(b):
            pltpu.make_async_copy(rows[b], acc.at[dstv.at[0]], ssem[b]).wait()

        def hist_batch(j):
            def hbody(k, _):
                d = dstv[j, pl.ds(16 * k, 16)]
                occ, last = plsc.scan_count(d)
                plsc.addupdate_scatter(
                    hist, [lax.shift_right_logical(d, 7),
                           lax.bitwise_and(d, 127)],
                    occ.astype(jnp.float32), mask=last)
                return 0

            lax.fori_loop(0, B // 16, hbody, 0)

        def issue(j, b, do_swait, do_gather):
            bw = (b + LEAD) % GB
            if do_swait:
                wait_s(bw)
            if do_gather:
                pltpu.async_copy(table.at[srcv.at[j + LEAD]], rows[bw], gsem[bw])
            wait_g(b, j)
            pltpu.async_copy(rows[b], acc.at[dstv.at[j]], ssem[b], add=True)
            if with_cnt:
                hist_batch(j)

        for b in range(LEAD):
            pltpu.async_copy(table.at[srcv.at[b]], rows[b], gsem[b])
        for b in range(GB):
            issue(b, b, do_swait=(b >= GB - LEAD), do_gather=True)

        def outer(t, _):
            for b in range(GB):
                issue(GB * t + b, b, do_swait=True, do_gather=True)
            return 0

        lax.fori_loop(1, NB // GB - 1, outer, 0)
        for b in range(GB):
            j = NB - GB + b
            issue(j, b, do_swait=True, do_gather=(j + LEAD < NB))
        for b in range(LEAD, GB):
            wait_s(b)
        if with_cnt:
            pltpu.sync_copy(hist, cacc.at[rowid.at[0]], add=True)
        plsc.subcore_barrier()

        for k in range(CH):
            sl = pl.ds(base + k * CO, CO)
            pltpu.sync_copy(acc.at[sl], zrows)
            pltpu.sync_copy(zrows, agg_out.at[sl, pl.ds(c * H, H)])
        if with_cnt:
            pltpu.sync_copy(cacc.at[pl.ds(s * HT, HT)], cstage)
            pltpu.sync_copy(cstage, cnt_out.at[c, pl.ds(s * HT, HT)])

    return pl.kernel(
        body, out_type=out_type, mesh=mesh, scratch_types=scratch,
        compiler_params=pltpu.CompilerParams(
            use_tc_tiling_on_sc=False,
            needs_layout_passes=False if with_cnt else None))


_sc_l1 = _make_sc(with_cnt=True)
_sc_l2 = _make_sc(with_cnt=False)


@jax.jit
def kernel(x, edge_index, W1l, b1l, W1r, W2l, b2l, W2r):
    idx3 = edge_index.astype(jnp.int32).reshape(2, NW, NB, B)

    y1, z1 = _tc_pre(x, W1l, W1r, b1l.reshape(1, H))
    agg1p, cntp = _sc_l1(y1, idx3)
    cnt2 = cntp.reshape(NC, NPAD)
    y2, z2 = _tc_mid(agg1p, cnt2, z1, W2l, W2r, b2l.reshape(1, H))
    [agg2p] = _sc_l2(y2, idx3)
    return _tc_post(agg2p, cnt2, z2)


# speedup vs baseline: 1.5633x; 1.0082x over previous
import jax
import jax.numpy as jnp
from jax import lax
from jax.experimental import pallas as pl
from jax.experimental.pallas import tpu as pltpu
from jax.experimental.pallas import tpu_sc as plsc

N = 10000
NPAD = 10240
F = 128
H = 64
E = 320000

NC = 2
NS = 16
NW = NC * NS
B = 80
EPW = E // NW
EPAD = EPW
NB = EPAD // B
RPT = NPAD // NS
CO = 80
CH = RPT // CO
GB = 5
HR = NPAD // 128
HT = HR // NS


RB = 2048
_GRID = (NPAD // RB,)
_rows = pl.BlockSpec((RB, H), lambda i: (i, 0))
_rows128 = pl.BlockSpec((RB, 2 * H), lambda i: (i, 0))
_cntspec = pl.BlockSpec((NC, RB), lambda i: (0, i))


def _fullspec(r, c):
    return pl.BlockSpec((r, c), lambda i: (0, 0))


def _pre_body(x_ref, wl_ref, wr_ref, b_ref, y_ref, z_ref):
    x = x_ref[...]
    y_ref[...] = jnp.dot(x, wl_ref[...], preferred_element_type=jnp.float32)
    z_ref[...] = jnp.dot(x, wr_ref[...], preferred_element_type=jnp.float32) + b_ref[...]


def _inv_cnt(cntp_ref):
    cnt = jnp.maximum(cntp_ref[0:1, :] + cntp_ref[1:2, :], 1.0)
    return jnp.transpose(cnt)


def _mid_body(aggp_ref, cntp_ref, z1_ref, wl_ref, wr_ref, b_ref, y_ref, z_ref):
    agg = aggp_ref[:, 0:H] + aggp_ref[:, H:2 * H]
    h = jax.nn.sigmoid(agg / _inv_cnt(cntp_ref) + z1_ref[...])
    y_ref[...] = jnp.dot(h, wl_ref[...], preferred_element_type=jnp.float32)
    z_ref[...] = jnp.dot(h, wr_ref[...], preferred_element_type=jnp.float32) + b_ref[...]


def _post_body(aggp_ref, cntp_ref, z2_ref, out_ref):
    agg = aggp_ref[:, 0:H] + aggp_ref[:, H:2 * H]
    out_ref[...] = agg / _inv_cnt(cntp_ref) + z2_ref[...]


_tc_pre = pl.pallas_call(
    _pre_body,
    grid=_GRID,
    in_specs=[pl.BlockSpec((RB, F), lambda i: (i, 0)),
              _fullspec(F, H), _fullspec(F, H), _fullspec(1, H)],
    out_specs=[_rows, _rows],
    out_shape=[jax.ShapeDtypeStruct((N, H), jnp.float32),
               jax.ShapeDtypeStruct((N, H), jnp.float32)],
)

_tc_mid = pl.pallas_call(
    _mid_body,
    grid=_GRID,
    in_specs=[_rows128, _cntspec, _rows,
              _fullspec(H, H), _fullspec(H, H), _fullspec(1, H)],
    out_specs=[_rows, _rows],
    out_shape=[jax.ShapeDtypeStruct((N, H), jnp.float32),
               jax.ShapeDtypeStruct((N, H), jnp.float32)],
)

_tc_post = pl.pallas_call(
    _post_body,
    grid=_GRID,
    in_specs=[_rows128, _cntspec, _rows],
    out_specs=_rows,
    out_shape=jax.ShapeDtypeStruct((N, H), jnp.float32),
)


def _fill_rows(buf, ncols, val):
    v = jnp.full((16,), val, jnp.float32)

    def body(i, _):
        for j in range(ncols // 16):
            buf[i, pl.ds(j * 16, 16)] = v
        return 0

    lax.fori_loop(0, buf.shape[0], body, 0)


def _make_sc(with_cnt):
    mesh = plsc.VectorSubcoreMesh(
        core_axis_name="c", subcore_axis_name="s", num_cores=NC, num_subcores=NS)
    out_type = [jax.ShapeDtypeStruct((NPAD, NC * H), jnp.float32)]
    scratch = (
        [pltpu.VMEM((NB, B), jnp.int32),
         pltpu.VMEM((NB, B), jnp.int32)]
        + [pltpu.VMEM((B, H), jnp.float32)] * GB
        + [pltpu.VMEM((CO, H), jnp.float32),
           pltpu.VMEM_SHARED((NPAD, H), jnp.float32)]
        + [pltpu.SemaphoreType.DMA] * GB
        + [pltpu.SemaphoreType.DMA] * GB
    )
    if with_cnt:
        out_type.append(jax.ShapeDtypeStruct((NC, HR, 128), jnp.float32))
        scratch = scratch + [
            pltpu.VMEM((HR, 128), jnp.float32),
            pltpu.VMEM((1, HR), jnp.int32),
            pltpu.VMEM((HT, 128), jnp.float32),
            pltpu.VMEM_SHARED((HR, 128), jnp.float32),
        ]

    def body(table, idx3, *refs):
        if with_cnt:
            (agg_out, cnt_out, srcv, dstv, *rest) = refs
        else:
            (agg_out, srcv, dstv, *rest) = refs
            cnt_out = None
        rows = rest[:GB]
        zrows, acc = rest[GB], rest[GB + 1]
        gsem = rest[GB + 2:2 * GB + 2]
        ssem = rest[2 * GB + 2:3 * GB + 2]
        if with_cnt:
            hist, rowid, cstage, cacc = rest[3 * GB + 2:]
        else:
            hist = rowid = cstage = cacc = None
        c = lax.axis_index("c")
        s = lax.axis_index("s")
        wid = c * NS + s
        base = s * RPT

        _fill_rows(zrows, H, 0.0)
        if with_cnt:
            _fill_rows(hist, 128, 0.0)
            for k in range(HR // 16):
                rowid[0, pl.ds(16 * k, 16)] = lax.iota(jnp.int32, 16) + 16 * k

            @pl.when(s == 0)
            def _():
                pltpu.sync_copy(hist, cacc)

        for k in range(CH):
            sl = pl.ds(base + k * CO, CO)
            pltpu.sync_copy(zrows, acc.at[sl])
        plsc.subcore_barrier()

        pltpu.sync_copy(idx3.at[0, wid], srcv)
        pltpu.sync_copy(idx3.at[1, wid], dstv)

        LEAD = GB - 2

        def wait_g(b, j):
            pltpu.make_async_copy(table.at[srcv.at[j]], rows[b], gsem[b]).wait()

        def wait_s(b):
            pltpu.make_async_copy(rows[b], acc.at[dstv.at[0]], ssem[b]).wait()

        def hist_batch(j):
            def hbody(k, _):
                d = dstv[j, pl.ds(16 * k, 16)]
                occ, last = plsc.scan_count(d)
                plsc.addupdate_scatter(
                    hist, [lax.shift_right_logical(d, 7),
                           lax.bitwise_and(d, 127)],
                    occ.astype(jnp.float32), mask=last)
                return 0

            lax.fori_loop(0, B // 16, hbody, 0)

        def issue(j, b, do_swait, do_gather):
            bw = (b + LEAD) % GB
            if do_swait:
                wait_s(bw)
            if do_gather:
                pltpu.async_copy(table.at[srcv.at[j + LEAD]], rows[bw], gsem[bw])
            wait_g(b, j)
            pltpu.async_copy(rows[b], acc.at[dstv.at[j]], ssem[b], add=True)
            if with_cnt:
                hist_batch(j)

        for b in range(LEAD):
            pltpu.async_copy(table.at[srcv.at[b]], rows[b], gsem[b])
        for b in range(GB):
            issue(b, b, do_swait=(b >= GB - LEAD), do_gather=True)

        def outer(t, _):
            for b in range(GB):
                issue(GB * t + b, b, do_swait=True, do_gather=True)
            return 0

        lax.fori_loop(1, NB // GB - 1, outer, 0)
        for b in range(GB):
            j = NB - GB + b
            issue(j, b, do_swait=True, do_gather=(j + LEAD < NB))
        for b in range(LEAD, GB):
            wait_s(b)
        if with_cnt:
            pltpu.sync_copy(hist, cacc.at[rowid.at[0]], add=True)
        plsc.subcore_barrier()

        for k in range(CH):
            sl = pl.ds(base + k * CO, CO)
            pltpu.sync_copy(acc.at[sl], zrows)
            pltpu.sync_copy(zrows, agg_out.at[sl, pl.ds(c * H, H)])
        if with_cnt:
            pltpu.sync_copy(cacc.at[pl.ds(s * HT, HT)], cstage)
            pltpu.sync_copy(cstage, cnt_out.at[c, pl.ds(s * HT, HT)])

    return pl.kernel(
        body, out_type=out_type, mesh=mesh, scratch_types=scratch,
        compiler_params=pltpu.CompilerParams(
            use_tc_tiling_on_sc=False,
            needs_layout_passes=False if with_cnt else None))


_sc_l1 = _make_sc(with_cnt=True)
_sc_l2 = _make_sc(with_cnt=False)


@jax.jit
def kernel(x, edge_index, W1l, b1l, W1r, W2l, b2l, W2r):
    idx3 = edge_index.astype(jnp.int32).reshape(2, NW, NB, B)

    y1, z1 = _tc_pre(x, W1l, W1r, b1l.reshape(1, H))
    agg1p, cntp = _sc_l1(y1, idx3)
    cnt2 = cntp.reshape(NC, NPAD)
    y2, z2 = _tc_mid(agg1p, cnt2, z1, W2l, W2r, b2l.reshape(1, H))
    [agg2p] = _sc_l2(y2, idx3)
    return _tc_post(agg2p, cnt2, z2)
